# baseline (speedup 1.0000x reference)
import jax
import jax.numpy as jnp
import numpy as np
from jax.experimental import pallas as pl
from jax.experimental.pallas import tpu as pltpu

H, W = 37, 75
NT = H * W
NP = 2944


def _build_masks() -> np.ndarray:
    t = np.arange(NP)
    w = t % W
    valid = t < NT
    rows = [
        valid.astype(np.float32),
        (valid & (w != W - 1)).astype(np.float32),
        (valid & (w != 0)).astype(np.float32),
    ]
    rows.extend(np.zeros(NP, np.float32) for _ in range(5))
    return np.stack(rows)


_MASKS = _build_masks()


def _shift(x, delta):
    if delta == 0:
        return x
    return jnp.concatenate([x[:, delta:], x[:, :delta]], axis=1)


def _conv3x3(x_bf, w_ref, masks_ref, mvalid, relu, out_bf16):
    ml = masks_ref[pl.ds(1, 1), :].astype(jnp.bfloat16)
    mr = masks_ref[pl.ds(2, 1), :].astype(jnp.bfloat16)
    xl = x_bf * ml
    xr = x_bf * mr
    acc = None
    for tap in range(9):
        di, dj = tap // 3 - 1, tap % 3 - 1
        src = x_bf if dj == 0 else (xr if dj == 1 else xl)
        xs = _shift(src, di * W + dj)
        y = jnp.dot(w_ref[0, tap], xs, preferred_element_type=jnp.float32)
        acc = y if acc is None else acc + y
    if relu:
        acc = jnp.maximum(acc, 0.0)
    if mvalid is not None:
        acc = acc * mvalid
    return acc.astype(jnp.bfloat16) if out_bf16 else acc


def _memory_body(qf_ref, keys_ref, masks_ref, w1_ref, w2_ref,
                 w3_ref, w4_ref, out_ref, x0_scratch, tq_scratch):
    b = pl.program_id(0)
    qt = qf_ref[...]
    qt = jnp.concatenate(
        [qt, jnp.zeros((qt.shape[0], NP - NT), jnp.float32)], axis=1)
    norm = jnp.sqrt(jnp.sum(qt * qt, axis=0, keepdims=True))
    qn = qt / jnp.maximum(norm, 1e-12)

    @pl.when(b == 0)
    def _theta_input():
        x0_scratch[...] = qn.astype(jnp.bfloat16)

    @pl.when(b == 1)
    def _memory_read():
        keys = keys_ref[...]
        k_norm = jnp.sqrt(jnp.sum(keys * keys, axis=1, keepdims=True))
        q_norm = jnp.sqrt(jnp.sum(qn * qn, axis=0, keepdims=True))
        dots = jnp.dot(keys, qn, preferred_element_type=jnp.float32)
        cos = dots / jnp.maximum(k_norm * q_norm, 1e-6)
        e = jnp.exp(cos - jnp.max(cos, axis=0, keepdims=True))
        score = e / jnp.sum(e, axis=0, keepdims=True)
        upd = jax.lax.dot_general(
            keys, score, (((0,), (0,)), ((), ())),
            preferred_element_type=jnp.float32)
        x0_scratch[...] = (upd * masks_ref[pl.ds(0, 1), :]).astype(jnp.bfloat16)

    mvalid = masks_ref[pl.ds(0, 1), :]
    x = x0_scratch[...]
    x = _conv3x3(x, w1_ref, masks_ref, mvalid, relu=True, out_bf16=True)
    x = _conv3x3(x, w2_ref, masks_ref, mvalid, relu=True, out_bf16=True)
    x = _conv3x3(x, w3_ref, masks_ref, mvalid, relu=True, out_bf16=True)
    x = _conv3x3(x, w4_ref, masks_ref, None, relu=False, out_bf16=False)

    @pl.when(b == 0)
    def _store_tq():
        tq_scratch[...] = x

    @pl.when(b == 1)
    def _combine():
        tq = tq_scratch[...]
        tk = x
        num = jnp.sum(tk * tq, axis=0, keepdims=True)
        den = jnp.maximum(
            jnp.sqrt(jnp.sum(tk * tk, axis=0, keepdims=True))
            * jnp.sqrt(jnp.sum(tq * tq, axis=0, keepdims=True)), 1e-6)
        out_ref[...] = ((num / den) * qn)[:, :NT]


def _tap_weights(w_theta, w_thetak):
    o, i = w_theta.shape[0], w_theta.shape[1]
    ws = jnp.stack([w_theta, w_thetak]).astype(jnp.bfloat16)
    ws = jnp.transpose(ws, (0, 3, 4, 1, 2))
    return ws.reshape(2, 9, o, i)


@jax.jit
def _run(query, keys, theta_w1, theta_w2, theta_w3, theta_w4,
         thetak_w1, thetak_w2, thetak_w3, thetak_w4):
    d = query.shape[1]
    qf = query.reshape(d, NT)
    masks = jnp.asarray(_MASKS)
    w1 = _tap_weights(theta_w1, thetak_w1)
    w2 = _tap_weights(theta_w2, thetak_w2)
    w3 = _tap_weights(theta_w3, thetak_w3)
    w4 = _tap_weights(theta_w4, thetak_w4)

    full = lambda shape: pl.BlockSpec(shape, lambda b: (0,) * len(shape))
    per_branch = lambda shape: pl.BlockSpec((1,) + shape[1:], lambda b: (b, 0, 0, 0))

    out = pl.pallas_call(
        _memory_body,
        grid=(2,),
        in_specs=[
            full((d, NT)),
            full((256, d)),
            full((8, NP)),
            per_branch(w1.shape),
            per_branch(w2.shape),
            per_branch(w3.shape),
            per_branch(w4.shape),
        ],
        out_specs=full((d, NT)),
        out_shape=jax.ShapeDtypeStruct((d, NT), jnp.float32),
        scratch_shapes=[
            pltpu.VMEM((d, NP), jnp.bfloat16),
            pltpu.VMEM((64, NP), jnp.float32),
        ],
    )(qf, keys, masks, w1, w2, w3, w4)

    cfeature = out.reshape(1, d, H, W)
    return keys, cfeature


def kernel(query, keys, theta_w1, theta_w2, theta_w3, theta_w4,
           thetak_w1, thetak_w2, thetak_w3, thetak_w4, train=False):
    return _run(query, keys, theta_w1, theta_w2, theta_w3, theta_w4,
                thetak_w1, thetak_w2, thetak_w3, thetak_w4)


# speedup vs baseline: 1.0377x; 1.0377x over previous
import jax
import jax.numpy as jnp
import numpy as np
from jax.experimental import pallas as pl
from jax.experimental.pallas import tpu as pltpu

H, W = 37, 75
NT = H * W
NP = 2944


def _build_masks() -> np.ndarray:
    t = np.arange(NP)
    w = t % W
    valid = t < NT
    rows = [
        valid.astype(np.float32),
        (valid & (w != W - 1)).astype(np.float32),
        (valid & (w != 0)).astype(np.float32),
    ]
    rows.extend(np.zeros(NP, np.float32) for _ in range(5))
    return np.stack(rows)


_MASKS = _build_masks()


def _shift(x, delta):
    if delta == 0:
        return x
    return jnp.concatenate([x[:, delta:], x[:, :delta]], axis=1)


def _conv3x3(x_bf, w_ref, masks_ref, mvalid, relu, out_bf16):
    ml = masks_ref[pl.ds(1, 1), :].astype(jnp.bfloat16)
    mr = masks_ref[pl.ds(2, 1), :].astype(jnp.bfloat16)
    xl = x_bf * ml
    xr = x_bf * mr
    acc = None
    for kw in range(3):
        dj = kw - 1
        src = x_bf if dj == 0 else (xr if dj == 1 else xl)
        x3 = jnp.concatenate(
            [_shift(src, dj - W), _shift(src, dj), _shift(src, dj + W)],
            axis=0)
        y = jnp.dot(w_ref[0, kw], x3, preferred_element_type=jnp.float32)
        acc = y if acc is None else acc + y
    if relu:
        acc = jnp.maximum(acc, 0.0)
    if mvalid is not None:
        acc = acc * mvalid
    return acc.astype(jnp.bfloat16) if out_bf16 else acc


def _memory_body(qf_ref, keys_ref, masks_ref, w1_ref, w2_ref,
                 w3_ref, w4_ref, out_ref, x0_scratch, tq_scratch):
    b = pl.program_id(0)
    qt = qf_ref[...]
    qt = jnp.concatenate(
        [qt, jnp.zeros((qt.shape[0], NP - NT), jnp.float32)], axis=1)
    norm = jnp.sqrt(jnp.sum(qt * qt, axis=0, keepdims=True))
    qn = qt / jnp.maximum(norm, 1e-12)

    @pl.when(b == 0)
    def _theta_input():
        x0_scratch[...] = qn.astype(jnp.bfloat16)

    @pl.when(b == 1)
    def _memory_read():
        keys = keys_ref[...]
        k_norm = jnp.sqrt(jnp.sum(keys * keys, axis=1, keepdims=True))
        q_norm = jnp.sqrt(jnp.sum(qn * qn, axis=0, keepdims=True))
        dots = jnp.dot(keys, qn, preferred_element_type=jnp.float32)
        cos = dots / jnp.maximum(k_norm * q_norm, 1e-6)
        e = jnp.exp(cos - jnp.max(cos, axis=0, keepdims=True))
        score = e / jnp.sum(e, axis=0, keepdims=True)
        upd = jax.lax.dot_general(
            keys, score, (((0,), (0,)), ((), ())),
            preferred_element_type=jnp.float32)
        x0_scratch[...] = (upd * masks_ref[pl.ds(0, 1), :]).astype(jnp.bfloat16)

    mvalid = masks_ref[pl.ds(0, 1), :]
    x = x0_scratch[...]
    x = _conv3x3(x, w1_ref, masks_ref, mvalid, relu=True, out_bf16=True)
    x = _conv3x3(x, w2_ref, masks_ref, mvalid, relu=True, out_bf16=True)
    x = _conv3x3(x, w3_ref, masks_ref, mvalid, relu=True, out_bf16=True)
    x = _conv3x3(x, w4_ref, masks_ref, None, relu=False, out_bf16=False)

    @pl.when(b == 0)
    def _store_tq():
        tq_scratch[...] = x

    @pl.when(b == 1)
    def _combine():
        tq = tq_scratch[...]
        tk = x
        num = jnp.sum(tk * tq, axis=0, keepdims=True)
        den = jnp.maximum(
            jnp.sqrt(jnp.sum(tk * tk, axis=0, keepdims=True))
            * jnp.sqrt(jnp.sum(tq * tq, axis=0, keepdims=True)), 1e-6)
        out_ref[...] = ((num / den) * qn)[:, :NT]


def _tap_weights(w_theta, w_thetak):
    o, i = w_theta.shape[0], w_theta.shape[1]
    ws = jnp.stack([w_theta, w_thetak]).astype(jnp.bfloat16)
    ws = jnp.transpose(ws, (0, 4, 1, 3, 2))
    return ws.reshape(2, 3, o, 3 * i)


@jax.jit
def _run(query, keys, theta_w1, theta_w2, theta_w3, theta_w4,
         thetak_w1, thetak_w2, thetak_w3, thetak_w4):
    d = query.shape[1]
    qf = query.reshape(d, NT)
    masks = jnp.asarray(_MASKS)
    w1 = _tap_weights(theta_w1, thetak_w1)
    w2 = _tap_weights(theta_w2, thetak_w2)
    w3 = _tap_weights(theta_w3, thetak_w3)
    w4 = _tap_weights(theta_w4, thetak_w4)

    full = lambda shape: pl.BlockSpec(shape, lambda b: (0,) * len(shape))
    per_branch = lambda shape: pl.BlockSpec((1,) + shape[1:], lambda b: (b, 0, 0, 0))

    out = pl.pallas_call(
        _memory_body,
        grid=(2,),
        in_specs=[
            full((d, NT)),
            full((256, d)),
            full((8, NP)),
            per_branch(w1.shape),
            per_branch(w2.shape),
            per_branch(w3.shape),
            per_branch(w4.shape),
        ],
        out_specs=full((d, NT)),
        out_shape=jax.ShapeDtypeStruct((d, NT), jnp.float32),
        scratch_shapes=[
            pltpu.VMEM((d, NP), jnp.bfloat16),
            pltpu.VMEM((64, NP), jnp.float32),
        ],
    )(qf, keys, masks, w1, w2, w3, w4)

    cfeature = out.reshape(1, d, H, W)
    return keys, cfeature


def kernel(query, keys, theta_w1, theta_w2, theta_w3, theta_w4,
           thetak_w1, thetak_w2, thetak_w3, thetak_w4, train=False):
    return _run(query, keys, theta_w1, theta_w2, theta_w3, theta_w4,
                thetak_w1, thetak_w2, thetak_w3, thetak_w4)


# speedup vs baseline: 1.0509x; 1.0127x over previous
import jax
import jax.numpy as jnp
import numpy as np
from jax.experimental import pallas as pl
from jax.experimental.pallas import tpu as pltpu

H, W = 37, 75
NT = H * W
NP = 2944


def _build_masks() -> np.ndarray:
    t = np.arange(NP)
    w = t % W
    valid = t < NT
    rows = [
        valid.astype(np.float32),
        (valid & (w != W - 1)).astype(np.float32),
        (valid & (w != 0)).astype(np.float32),
    ]
    rows.extend(np.zeros(NP, np.float32) for _ in range(5))
    return np.stack(rows)


_MASKS = _build_masks()


def _shift(x, delta):
    if delta == 0:
        return x
    return jnp.concatenate([x[:, delta:], x[:, :delta]], axis=1)


def _conv3x3(x_bf, w_ref, masks_ref, mvalid, relu, out_bf16):
    ml = masks_ref[pl.ds(1, 1), :].astype(jnp.bfloat16)
    mr = masks_ref[pl.ds(2, 1), :].astype(jnp.bfloat16)
    xl = x_bf * ml
    xr = x_bf * mr
    parts = []
    for kw in range(3):
        dj = kw - 1
        src = x_bf if dj == 0 else (xr if dj == 1 else xl)
        parts.extend(
            [_shift(src, dj - W), _shift(src, dj), _shift(src, dj + W)])
    x9 = jnp.concatenate(parts, axis=0)
    acc = jnp.dot(w_ref[0, 0], x9, preferred_element_type=jnp.float32)
    if relu:
        acc = jnp.maximum(acc, 0.0)
    if mvalid is not None:
        acc = acc * mvalid
    return acc.astype(jnp.bfloat16) if out_bf16 else acc


def _memory_body(qf_ref, keys_ref, masks_ref, w1_ref, w2_ref,
                 w3_ref, w4_ref, out_ref, x0_scratch, tq_scratch):
    b = pl.program_id(0)
    qt = qf_ref[...]
    qt = jnp.concatenate(
        [qt, jnp.zeros((qt.shape[0], NP - NT), jnp.float32)], axis=1)
    norm = jnp.sqrt(jnp.sum(qt * qt, axis=0, keepdims=True))
    qn = qt / jnp.maximum(norm, 1e-12)

    @pl.when(b == 0)
    def _theta_input():
        x0_scratch[...] = qn.astype(jnp.bfloat16)

    @pl.when(b == 1)
    def _memory_read():
        keys = keys_ref[...]
        k_norm = jnp.sqrt(jnp.sum(keys * keys, axis=1, keepdims=True))
        q_norm = jnp.sqrt(jnp.sum(qn * qn, axis=0, keepdims=True))
        dots = jnp.dot(keys, qn, preferred_element_type=jnp.float32)
        cos = dots / jnp.maximum(k_norm * q_norm, 1e-6)
        e = jnp.exp(cos - jnp.max(cos, axis=0, keepdims=True))
        score = e / jnp.sum(e, axis=0, keepdims=True)
        upd = jax.lax.dot_general(
            keys, score, (((0,), (0,)), ((), ())),
            preferred_element_type=jnp.float32)
        x0_scratch[...] = (upd * masks_ref[pl.ds(0, 1), :]).astype(jnp.bfloat16)

    mvalid = masks_ref[pl.ds(0, 1), :]
    x = x0_scratch[...]
    x = _conv3x3(x, w1_ref, masks_ref, mvalid, relu=True, out_bf16=True)
    x = _conv3x3(x, w2_ref, masks_ref, mvalid, relu=True, out_bf16=True)
    x = _conv3x3(x, w3_ref, masks_ref, mvalid, relu=True, out_bf16=True)
    x = _conv3x3(x, w4_ref, masks_ref, None, relu=False, out_bf16=False)

    @pl.when(b == 0)
    def _store_tq():
        tq_scratch[...] = x

    @pl.when(b == 1)
    def _combine():
        tq = tq_scratch[...]
        tk = x
        num = jnp.sum(tk * tq, axis=0, keepdims=True)
        den = jnp.maximum(
            jnp.sqrt(jnp.sum(tk * tk, axis=0, keepdims=True))
            * jnp.sqrt(jnp.sum(tq * tq, axis=0, keepdims=True)), 1e-6)
        out_ref[...] = ((num / den) * qn)[:, :NT]


def _tap_weights(w_theta, w_thetak):
    o, i = w_theta.shape[0], w_theta.shape[1]
    ws = jnp.stack([w_theta, w_thetak]).astype(jnp.bfloat16)
    ws = jnp.transpose(ws, (0, 1, 4, 3, 2))
    return ws.reshape(2, 1, o, 9 * i)


@jax.jit
def _run(query, keys, theta_w1, theta_w2, theta_w3, theta_w4,
         thetak_w1, thetak_w2, thetak_w3, thetak_w4):
    d = query.shape[1]
    qf = query.reshape(d, NT)
    masks = jnp.asarray(_MASKS)
    w1 = _tap_weights(theta_w1, thetak_w1)
    w2 = _tap_weights(theta_w2, thetak_w2)
    w3 = _tap_weights(theta_w3, thetak_w3)
    w4 = _tap_weights(theta_w4, thetak_w4)

    full = lambda shape: pl.BlockSpec(shape, lambda b: (0,) * len(shape))
    per_branch = lambda shape: pl.BlockSpec((1,) + shape[1:], lambda b: (b, 0, 0, 0))

    out = pl.pallas_call(
        _memory_body,
        grid=(2,),
        in_specs=[
            full((d, NT)),
            full((256, d)),
            full((8, NP)),
            per_branch(w1.shape),
            per_branch(w2.shape),
            per_branch(w3.shape),
            per_branch(w4.shape),
        ],
        out_specs=full((d, NT)),
        out_shape=jax.ShapeDtypeStruct((d, NT), jnp.float32),
        scratch_shapes=[
            pltpu.VMEM((d, NP), jnp.bfloat16),
            pltpu.VMEM((64, NP), jnp.float32),
        ],
    )(qf, keys, masks, w1, w2, w3, w4)

    cfeature = out.reshape(1, d, H, W)
    return keys, cfeature


def kernel(query, keys, theta_w1, theta_w2, theta_w3, theta_w4,
           thetak_w1, thetak_w2, thetak_w3, thetak_w4, train=False):
    return _run(query, keys, theta_w1, theta_w2, theta_w3, theta_w4,
                thetak_w1, thetak_w2, thetak_w3, thetak_w4)
